# baseline (speedup 1.0000x reference)
import functools

import jax
import jax.numpy as jnp
from jax import lax
from jax.experimental import pallas as pl
from jax.experimental.pallas import tpu as pltpu
from jax.experimental.pallas import tpu_sc as plsc

_DIM = 256
_N_EMBED = 8192
_BETA = 0.25

_B = 16
_P = 1024
_KB = 512
_NK = _N_EMBED // _KB


_SPANS = ((0, 2736), (2736, 5472), (5472, 8192))


_ROW_ID_BASE = 0x3F800000


def _dist_argmin_body(zn_ref, en_ref, rid_ref, z_ref, emb2_ref, idx_ref):
    zb = z_ref[0]
    best_v = None
    best_i = None
    for lo, hi in _SPANS:
        eb2 = emb2_ref[lo:hi, :]
        en = en_ref[lo:hi, :]
        mm2 = lax.dot_general(eb2, zb, (((1,), (0,)), ((), ())),
                              preferred_element_type=jnp.float32)
        score = (zn_ref[0] + en) - mm2
        m = jnp.min(score, axis=0)
        am_f = jnp.min(jnp.where(score == m[None, :], rid_ref[lo:hi, :],
                                 jnp.float32(2.0)), axis=0)
        am = lax.bitcast_convert_type(am_f, jnp.int32) - _ROW_ID_BASE
        mr = m.astype(jnp.bfloat16).astype(jnp.float32)
        if best_v is None:
            best_v, best_i = mr, am
        else:
            better = m < best_v
            best_v = jnp.where(better, mr, best_v)
            best_i = jnp.where(better, am, best_i)
    idx_ref[...] = best_i.reshape(1, 1, _P)


def _dist_argmin(zn3, en2, rid2, z3, emb2):
    return pl.pallas_call(
        _dist_argmin_body,
        grid=(_B,),
        in_specs=[
            pl.BlockSpec((1, 1, _P), lambda b: (b, 0, 0)),
            pl.BlockSpec((_N_EMBED, 1), lambda b: (0, 0)),
            pl.BlockSpec((_N_EMBED, 1), lambda b: (0, 0)),
            pl.BlockSpec((1, _DIM, _P), lambda b: (b, 0, 0)),
            pl.BlockSpec((_N_EMBED, _DIM), lambda b: (0, 0)),
        ],
        out_specs=pl.BlockSpec((1, 1, _P), lambda b: (b, 0, 0)),
        out_shape=jax.ShapeDtypeStruct((_B, 1, _P), jnp.int32),
        compiler_params=pltpu.CompilerParams(
            dimension_semantics=("parallel",)),
    )(zn3, en2, rid2, z3, emb2)


_NW = 32
_CH = 128


@functools.lru_cache(maxsize=4)
def _make_sc_gather(n_tokens):
    bpw = n_tokens // _NW
    nch = bpw // _CH

    def body(table_hbm, idx_hbm, out_hbm, idx_v, rows_v, sem):
        wid = lax.axis_index("s") * 2 + lax.axis_index("c")
        base = wid * bpw
        for j in range(nch):
            off = base + j * _CH
            pltpu.sync_copy(idx_hbm.at[pl.ds(off, _CH)], idx_v)
            pltpu.async_copy(table_hbm.at[idx_v], rows_v, sem).wait()
            pltpu.sync_copy(rows_v, out_hbm.at[pl.ds(off, _CH)])

    return pl.kernel(
        body,
        out_type=jax.ShapeDtypeStruct((n_tokens, _DIM), jnp.float32),
        mesh=plsc.VectorSubcoreMesh(core_axis_name="c", subcore_axis_name="s"),
        scratch_types=[
            pltpu.VMEM((_CH,), jnp.int32),
            pltpu.VMEM((_CH, _DIM), jnp.float32),
            pltpu.SemaphoreType.DMA,
        ],
    )


def _finalize_body(zq_ref, z_ref, o_ref, ls_ref):
    zqt = zq_ref[0].T
    dd = zqt - z_ref[0]
    o_ref[0] = z_ref[0] + dd
    ls_ref[...] = jnp.sum(dd * dd).reshape(1, 1, 1)


def _finalize(zq3, z3):
    return pl.pallas_call(
        _finalize_body,
        grid=(_B,),
        in_specs=[
            pl.BlockSpec((1, _P, _DIM), lambda b: (b, 0, 0)),
            pl.BlockSpec((1, _DIM, _P), lambda b: (b, 0, 0)),
        ],
        out_specs=[
            pl.BlockSpec((1, _DIM, _P), lambda b: (b, 0, 0)),
            pl.BlockSpec((1, 1, 1), lambda b: (b, 0, 0)),
        ],
        out_shape=[
            jax.ShapeDtypeStruct((_B, _DIM, _P), jnp.float32),
            jax.ShapeDtypeStruct((_B, 1, 1), jnp.float32),
        ],
        compiler_params=pltpu.CompilerParams(
            dimension_semantics=("parallel",)),
    )(zq3, z3)


def kernel(z, emb_weight):
    z3 = z.reshape(_B, _DIM, _P)
    zf = jnp.transpose(z, (0, 2, 3, 1)).reshape(-1, _DIM)
    zn3 = jnp.sum(zf ** 2, axis=1).reshape(_B, 1, _P)
    en2 = jnp.sum(emb_weight ** 2, axis=1).reshape(_N_EMBED, 1)
    rid2 = lax.bitcast_convert_type(
        jnp.arange(_N_EMBED, dtype=jnp.int32) + jnp.int32(_ROW_ID_BASE),
        jnp.float32).reshape(_N_EMBED, 1)

    emb2 = (emb_weight + emb_weight).astype(jnp.bfloat16)
    idx3 = _dist_argmin(zn3, en2, rid2, z3.astype(jnp.bfloat16), emb2)
    idx = idx3.reshape(_B * _P)

    zq = _make_sc_gather(_B * _P)(emb_weight, idx)

    out3, ls = _finalize(zq.reshape(_B, _P, _DIM), z3)
    out = out3.reshape(_B, _DIM, 32, 32)
    loss = (jnp.sum(ls) * ((1.0 + _BETA) / (_B * _P * _DIM))).reshape(())
    return (out, loss, idx)


# speedup vs baseline: 1.0401x; 1.0401x over previous
import functools

import jax
import jax.numpy as jnp
from jax import lax
from jax.experimental import pallas as pl
from jax.experimental.pallas import tpu as pltpu
from jax.experimental.pallas import tpu_sc as plsc

_DIM = 256
_N_EMBED = 8192
_BETA = 0.25

_B = 16
_P = 1024
_KB = 512
_NK = _N_EMBED // _KB


_SPANS = ((0, 2736), (2736, 5472), (5472, 8192))


_ROW_ID_BASE = 0x3F800000


def _dist_argmin_body(zn_ref, en_ref, rid_ref, z_ref, emb2_ref, idx_ref):
    zb = z_ref[0]
    best_v = None
    best_i = None
    for lo, hi in _SPANS:
        eb2 = emb2_ref[lo:hi, :]
        en = en_ref[lo:hi, :]
        mm2 = lax.dot_general(eb2, zb, (((1,), (0,)), ((), ())),
                              preferred_element_type=jnp.float32)
        score = (zn_ref[0] + en) - mm2
        m = jnp.min(score, axis=0)
        am_f = jnp.min(jnp.where(score == m[None, :], rid_ref[lo:hi, :],
                                 jnp.float32(2.0)), axis=0)
        am = lax.bitcast_convert_type(am_f, jnp.int32) - _ROW_ID_BASE
        mr = m.astype(jnp.bfloat16).astype(jnp.float32)
        if best_v is None:
            best_v, best_i = mr, am
        else:
            better = m < best_v
            best_v = jnp.where(better, mr, best_v)
            best_i = jnp.where(better, am, best_i)
    idx_ref[...] = best_i.reshape(1, 1, _P)


def _dist_argmin(zn3, en2, rid2, z3, emb2):
    return pl.pallas_call(
        _dist_argmin_body,
        grid=(_B,),
        in_specs=[
            pl.BlockSpec((1, 1, _P), lambda b: (b, 0, 0)),
            pl.BlockSpec((_N_EMBED, 1), lambda b: (0, 0)),
            pl.BlockSpec((_N_EMBED, 1), lambda b: (0, 0)),
            pl.BlockSpec((1, _DIM, _P), lambda b: (b, 0, 0)),
            pl.BlockSpec((_N_EMBED, _DIM), lambda b: (0, 0)),
        ],
        out_specs=pl.BlockSpec((1, 1, _P), lambda b: (b, 0, 0)),
        out_shape=jax.ShapeDtypeStruct((_B, 1, _P), jnp.int32),
        compiler_params=pltpu.CompilerParams(
            dimension_semantics=("parallel",)),
    )(zn3, en2, rid2, z3, emb2)


_NW = 32
_CH = 128


@functools.lru_cache(maxsize=4)
def _make_sc_gather(n_tokens):
    bpw = n_tokens // _NW
    nch = bpw // _CH

    def body(table_hbm, idx_hbm, out_hbm, idx_v, rows_v, sem):
        wid = lax.axis_index("s") * 2 + lax.axis_index("c")
        base = wid * bpw
        for j in range(nch):
            off = base + j * _CH
            pltpu.sync_copy(idx_hbm.at[pl.ds(off, _CH)], idx_v)
            pltpu.async_copy(table_hbm.at[idx_v], rows_v, sem).wait()
            pltpu.sync_copy(rows_v, out_hbm.at[pl.ds(off, _CH)])

    return pl.kernel(
        body,
        out_type=jax.ShapeDtypeStruct((n_tokens, _DIM), jnp.float32),
        mesh=plsc.VectorSubcoreMesh(core_axis_name="c", subcore_axis_name="s"),
        scratch_types=[
            pltpu.VMEM((_CH,), jnp.int32),
            pltpu.VMEM((_CH, _DIM), jnp.float32),
            pltpu.SemaphoreType.DMA,
        ],
    )


def _finalize_body(zq_ref, z_ref, o_ref, ls_ref):
    zqt = zq_ref[0].T
    dd = zqt - z_ref[0]
    o_ref[0] = z_ref[0] + dd
    ls_ref[...] = jnp.sum(dd * dd).reshape(1, 1, 1)


def _finalize(zq3, z3):
    return pl.pallas_call(
        _finalize_body,
        grid=(_B,),
        in_specs=[
            pl.BlockSpec((1, _P, _DIM), lambda b: (b, 0, 0)),
            pl.BlockSpec((1, _DIM, _P), lambda b: (b, 0, 0)),
        ],
        out_specs=[
            pl.BlockSpec((1, _DIM, _P), lambda b: (b, 0, 0)),
            pl.BlockSpec((1, 1, 1), lambda b: (b, 0, 0)),
        ],
        out_shape=[
            jax.ShapeDtypeStruct((_B, _DIM, _P), jnp.float32),
            jax.ShapeDtypeStruct((_B, 1, 1), jnp.float32),
        ],
        compiler_params=pltpu.CompilerParams(
            dimension_semantics=("parallel",)),
    )(zq3, z3)


def kernel(z, emb_weight):
    z3 = z.reshape(_B, _DIM, _P)
    zf = jnp.transpose(z, (0, 2, 3, 1)).reshape(-1, _DIM)
    zn3 = jnp.sum(zf ** 2, axis=1).reshape(_B, 1, _P)
    en2 = jnp.sum(emb_weight ** 2, axis=1).reshape(_N_EMBED, 1)
    rid2 = lax.bitcast_convert_type(
        jnp.arange(_N_EMBED, dtype=jnp.int32) + jnp.int32(_ROW_ID_BASE),
        jnp.float32).reshape(_N_EMBED, 1)

    emb2 = emb_weight + emb_weight
    idx3 = _dist_argmin(zn3, en2, rid2, z3, emb2)
    idx = idx3.reshape(_B * _P)

    zq = _make_sc_gather(_B * _P)(emb_weight, idx)

    out3, ls = _finalize(zq.reshape(_B, _P, _DIM), z3)
    out = out3.reshape(_B, _DIM, 32, 32)
    loss = (jnp.sum(ls) * ((1.0 + _BETA) / (_B * _P * _DIM))).reshape(())
    return (out, loss, idx)
